# Initial kernel scaffold; baseline (speedup 1.0000x reference)
import jax
import jax.numpy as jnp
from jax.experimental import pallas as pl


def kernel(features, table0, table1, table2, Wh, bh, Wo, bo):
    raise NotImplementedError("write your pallas kernel here")


# speedup vs baseline: 7.5541x; 7.5541x over previous
import functools

import jax
import jax.numpy as jnp
from jax.experimental import pallas as pl
from jax.experimental.pallas import tpu as pltpu
from jax.experimental.pallas import tpu_sc as plsc

_EMB = 32
_W = 128
_BM = 2048
_LINEAR_IN = 832
_NPAD = 128


def _sc_gather(table0, idx0, table1, idx1, table2, idx2):
    n0, n1, n2 = idx0.shape[1], idx1.shape[1], idx2.shape[1]
    mesh = plsc.VectorSubcoreMesh(core_axis_name="core", subcore_axis_name="subcore")

    @functools.partial(
        pl.kernel,
        out_type=[
            jax.ShapeDtypeStruct((n0, _EMB), jnp.float32),
            jax.ShapeDtypeStruct((n1, _EMB), jnp.float32),
            jax.ShapeDtypeStruct((n2, _EMB), jnp.float32),
        ],
        mesh=mesh,
        compiler_params=pltpu.CompilerParams(use_tc_tiling_on_sc=False),
    )
    def gather_kernel(t0_hbm, i0_hbm, t1_hbm, i1_hbm, t2_hbm, i2_hbm,
                      g0_hbm, g1_hbm, g2_hbm):
        def run(table_hbm, i_hbm, o_hbm, n):
            def body(i_vmem, o_vmem):
                pltpu.sync_copy(table_hbm.at[i_vmem.at[0]], o_vmem)

            pltpu.emit_pipeline(
                body,
                grid=(n // _W,),
                in_specs=[pl.BlockSpec((1, _W), index_map=lambda i: (0, i))],
                out_specs=[pl.BlockSpec((_W, _EMB), index_map=lambda i: (i, 0))],
                core_axis_name=("core", "subcore"),
                dimension_semantics=(pltpu.PARALLEL,),
            )(i_hbm, o_hbm)

        run(t0_hbm, i0_hbm, g0_hbm, n0)
        run(t1_hbm, i1_hbm, g1_hbm, n1)
        run(t2_hbm, i2_hbm, g2_hbm, n2)

    return gather_kernel(table0, idx0, table1, idx1, table2, idx2)


def _mlp_body(x0_ref, x1_ref, x2_ref, whT_ref, bh_ref, woT_ref, bo_ref, out_ref):
    h = jnp.dot(x0_ref[...], whT_ref[0:640, :], preferred_element_type=jnp.float32)
    h = h + jnp.dot(x1_ref[...], whT_ref[640:800, :], preferred_element_type=jnp.float32)
    h = h + jnp.dot(x2_ref[...], whT_ref[800:832, :], preferred_element_type=jnp.float32)
    h = jnp.maximum(h + bh_ref[...], 0.0)
    out_ref[...] = jnp.dot(h, woT_ref[...], preferred_element_type=jnp.float32) + bo_ref[...]


def _tc_mlp(x0, x1, x2, whT, bh, woT, bo):
    b = x0.shape[0]
    grid = (b // _BM,)
    return pl.pallas_call(
        _mlp_body,
        grid=grid,
        in_specs=[
            pl.BlockSpec((_BM, 20 * _EMB), lambda i: (i, 0)),
            pl.BlockSpec((_BM, 5 * _EMB), lambda i: (i, 0)),
            pl.BlockSpec((_BM, 1 * _EMB), lambda i: (i, 0)),
            pl.BlockSpec((_LINEAR_IN, _NPAD), lambda i: (0, 0)),
            pl.BlockSpec((1, _NPAD), lambda i: (0, 0)),
            pl.BlockSpec((_NPAD, _NPAD), lambda i: (0, 0)),
            pl.BlockSpec((1, _NPAD), lambda i: (0, 0)),
        ],
        out_specs=pl.BlockSpec((_BM, _NPAD), lambda i: (i, 0)),
        out_shape=jax.ShapeDtypeStruct((b, _NPAD), jnp.float32),
    )(x0, x1, x2, whT, bh, woT, bo)


def kernel(features, table0, table1, table2, Wh, bh, Wo, bo):
    b = features.shape[0]
    feats = features.astype(jnp.int32)
    idx0 = feats[:, 0:20].reshape(1, -1)
    idx1 = feats[:, 20:25].reshape(1, -1)
    idx2 = feats[:, 25:26].reshape(1, -1)

    g0, g1, g2 = _sc_gather(table0, idx0, table1, idx1, table2, idx2)
    x0 = g0.reshape(b, 20 * _EMB)
    x1 = g1.reshape(b, 5 * _EMB)
    x2 = g2.reshape(b, 1 * _EMB)

    hid = Wh.shape[0]
    whT = jnp.pad(Wh.T, ((0, 0), (0, _NPAD - hid)))
    bhp = jnp.pad(bh, (0, _NPAD - hid)).reshape(1, _NPAD)
    woT = jnp.pad(Wo.T, ((0, _NPAD - hid), (0, _NPAD - Wo.shape[0])))
    bop = jnp.pad(bo, (0, _NPAD - Wo.shape[0])).reshape(1, _NPAD)

    out = _tc_mlp(x0, x1, x2, whT, bhp, woT, bop)
    return out[:, : Wo.shape[0]].reshape(b, 1, Wo.shape[0])
